# baseline (speedup 1.0000x reference)
import functools

import jax
import jax.numpy as jnp
from jax.experimental import pallas as pl
from jax.experimental.pallas import tpu as pltpu


def _fused_body(a_ref, p1_ref, p2_ref, p3_ref, o_ref,
                y_scr, s_scr, sq_scr, w_scr, bu_scr, bk_scr, *,
                K, U, L, C_in, C2, M, Mt, eps):
    f32 = jnp.float32
    bf16 = jnp.bfloat16
    LC = L * C_in
    LC2 = L * C2
    KU = K * U
    s = pl.program_id(0)
    t = s % 2

    @pl.when(s == 0)
    def _init():
        i0 = jax.lax.broadcasted_iota(jnp.int32, (128, 128), 0)
        i1 = jax.lax.broadcasted_iota(jnp.int32, (128, 128), 1)
        bu_scr[...] = jnp.where(i0 // U == i1 // U, 1.0 / U, 0.0).astype(bf16)
        bk_scr[...] = jnp.where((i0 // KU == i1 // KU) & (i0 % U == i1 % U),
                                1.0 / K, 0.0).astype(bf16)
        p_all = jnp.concatenate(
            [p1_ref[...].reshape(LC2, C_in),
             0.1 * p2_ref[...].reshape(LC2, C_in),
             0.1 * p3_ref[...].reshape(LC2, C_in)], axis=1)
        c0 = jax.lax.broadcasted_iota(jnp.int32, (3 * C_in, 3 * LC), 0)
        c1 = jax.lax.broadcasted_iota(jnp.int32, (3 * C_in, 3 * LC), 1)
        sel = jnp.where(c0 == (c1 // LC) * C_in + c1 % C_in, 1.0, 0.0).astype(f32)
        w0 = jax.lax.broadcasted_iota(jnp.int32, (LC2, 3 * LC), 0)
        w1 = jax.lax.broadcasted_iota(jnp.int32, (LC2, 3 * LC), 1)
        mask = (w0 // C2 == (w1 % LC) // C_in).astype(f32)
        w_scr[...] = (jnp.dot(p_all, sel, preferred_element_type=f32)
                      * mask).astype(bf16)
        s_scr[...] = jnp.zeros_like(s_scr)
        sq_scr[...] = jnp.zeros_like(sq_scr)

    @pl.when(s < 2)
    def _compute():
        a = a_ref[...].reshape(LC, Mt)
        a2 = a.reshape(LC * Mt // 128, 128).astype(bf16)
        mean_u = jnp.dot(a2, bu_scr[...],
                         preferred_element_type=f32).reshape(LC, Mt)
        mean_k = jnp.dot(a2, bk_scr[...],
                         preferred_element_type=f32).reshape(LC, Mt)
        cat = jnp.concatenate([a.astype(bf16), mean_u.astype(bf16),
                               mean_k.astype(bf16)], axis=0)
        y = jnp.maximum(jnp.dot(w_scr[...], cat, preferred_element_type=f32),
                        0.0)
        f0 = jax.lax.broadcasted_iota(jnp.int32, (C2, LC2), 0)
        f1 = jax.lax.broadcasted_iota(jnp.int32, (C2, LC2), 1)
        fold = (f1 % C2 == f0).astype(bf16)
        yb = y.astype(bf16)
        s_scr[...] += jnp.dot(fold, yb, preferred_element_type=f32).sum(
            axis=-1, keepdims=True)
        sq_scr[...] += jnp.dot(fold, yb * yb, preferred_element_type=f32).sum(
            axis=-1, keepdims=True)
        y_scr[t] = y.reshape(L, C2, Mt)

    @pl.when(s >= 2)
    def _normalize():
        n = float(L * M)
        mu = s_scr[...] / n
        inv = jax.lax.rsqrt(sq_scr[...] / n - mu * mu + eps)
        o_ref[...] = (y_scr[t] - mu[None, :, :]) * inv[None, :, :]


def kernel(A_lcm, P1, P2, P3, ru, bu, rk, bk):
    L, C_in, M = A_lcm.shape
    C2 = P1.shape[1]
    U = M // ru.shape[1]
    K = M // rk.shape[1]
    Mt = M // 2
    body = functools.partial(_fused_body, K=K, U=U, L=L, C_in=C_in, C2=C2,
                             M=M, Mt=Mt, eps=1e-5)
    return pl.pallas_call(
        body,
        out_shape=jax.ShapeDtypeStruct((L, C2, M), jnp.float32),
        grid=(4,),
        in_specs=[
            pl.BlockSpec((L, C_in, Mt),
                         lambda s: (0, 0, jnp.where(s < 2, s, 1))),
            pl.BlockSpec((L, C2, C_in), lambda s: (0, 0, 0)),
            pl.BlockSpec((L, C2, C_in), lambda s: (0, 0, 0)),
            pl.BlockSpec((L, C2, C_in), lambda s: (0, 0, 0)),
        ],
        out_specs=pl.BlockSpec((L, C2, Mt),
                               lambda s: (0, 0, jnp.where(s >= 2, s - 2, 0))),
        scratch_shapes=[
            pltpu.VMEM((2, L, C2, M // 2), jnp.float32),
            pltpu.VMEM((C2, 1), jnp.float32),
            pltpu.VMEM((C2, 1), jnp.float32),
            pltpu.VMEM((L * C2, 3 * L * C_in), jnp.bfloat16),
            pltpu.VMEM((128, 128), jnp.bfloat16),
            pltpu.VMEM((128, 128), jnp.bfloat16),
        ],
        compiler_params=pltpu.CompilerParams(
            dimension_semantics=("arbitrary",),
            vmem_limit_bytes=48 << 20),
    )(A_lcm, P1, P2, P3)


# speedup vs baseline: 1.0435x; 1.0435x over previous
import functools

import jax
import jax.numpy as jnp
from jax.experimental import pallas as pl
from jax.experimental.pallas import tpu as pltpu


def _fused_body(a_ref, p1_ref, p2_ref, p3_ref, o_ref, *, K, U, C_in, eps):
    f32 = jnp.float32
    LC, M = a_ref.shape
    L, C2, _ = o_ref.shape
    LC2 = L * C2
    KU = K * U

    bf16 = jnp.bfloat16
    a = a_ref[...]

    a2 = a.reshape(LC * (M // 128), 128).astype(bf16)
    i0 = jax.lax.broadcasted_iota(jnp.int32, (128, 128), 0)
    i1 = jax.lax.broadcasted_iota(jnp.int32, (128, 128), 1)
    bu = jnp.where(i0 // U == i1 // U, 1.0 / U, 0.0).astype(bf16)
    bk = jnp.where((i0 // KU == i1 // KU) & (i0 % U == i1 % U),
                   1.0 / K, 0.0).astype(bf16)
    mean_u = jnp.dot(a2, bu, preferred_element_type=f32).reshape(LC, M)
    mean_k = jnp.dot(a2, bk, preferred_element_type=f32).reshape(LC, M)

    p_all = jnp.concatenate(
        [p1_ref[...].reshape(LC2, C_in),
         0.1 * p2_ref[...].reshape(LC2, C_in),
         0.1 * p3_ref[...].reshape(LC2, C_in)], axis=1)
    c0 = jax.lax.broadcasted_iota(jnp.int32, (3 * C_in, 3 * LC), 0)
    c1 = jax.lax.broadcasted_iota(jnp.int32, (3 * C_in, 3 * LC), 1)
    sel = jnp.where(c0 == (c1 // LC) * C_in + c1 % C_in, 1.0, 0.0).astype(f32)
    w0 = jax.lax.broadcasted_iota(jnp.int32, (LC2, 3 * LC), 0)
    w1 = jax.lax.broadcasted_iota(jnp.int32, (LC2, 3 * LC), 1)
    mask = (w0 // C2 == (w1 % LC) // C_in).astype(f32)
    W = (jnp.dot(p_all, sel, preferred_element_type=f32) * mask).astype(bf16)

    cat = jnp.concatenate([a.astype(bf16), mean_u.astype(bf16),
                           mean_k.astype(bf16)], axis=0)
    y = jnp.dot(W, cat, preferred_element_type=f32)
    y = jnp.maximum(y, 0.0)

    f0 = jax.lax.broadcasted_iota(jnp.int32, (C2, LC2), 0)
    f1 = jax.lax.broadcasted_iota(jnp.int32, (C2, LC2), 1)
    fold = (f1 % C2 == f0).astype(bf16)
    yb = y.astype(bf16)
    sy = jnp.dot(fold, yb, preferred_element_type=f32)
    sz = jnp.dot(fold, yb * yb, preferred_element_type=f32)
    n = float(L * M)
    mu = sy.sum(axis=-1, keepdims=True) / n
    msq = sz.sum(axis=-1, keepdims=True) / n
    inv = jax.lax.rsqrt(msq - mu * mu + eps)
    y3 = y.reshape(L, C2, M)
    o_ref[...] = (y3 - mu[None, :, :]) * inv[None, :, :]


def kernel(A_lcm, P1, P2, P3, ru, bu, rk, bk):
    L, C_in, M = A_lcm.shape
    C2 = P1.shape[1]
    U = M // ru.shape[1]
    K = M // rk.shape[1]
    return pl.pallas_call(
        functools.partial(_fused_body, K=K, U=U, C_in=C_in, eps=1e-5),
        out_shape=jax.ShapeDtypeStruct((L, C2, M), jnp.float32),
        compiler_params=pltpu.CompilerParams(
            vmem_limit_bytes=48 << 20),
    )(A_lcm.reshape(L * C_in, M), P1, P2, P3)


# speedup vs baseline: 1.0871x; 1.0417x over previous
import functools

import jax
import jax.numpy as jnp
from jax.experimental import pallas as pl
from jax.experimental.pallas import tpu as pltpu


def _fused_body(a_ref, p1_ref, p2_ref, p3_ref, o_ref, ob_scr, sems, *,
                K, U, NT, eps):
    f32 = jnp.float32
    L, C_in, M = a_ref.shape
    _, C2, _ = o_ref.shape
    LC = L * C_in
    LC2 = L * C2
    KU = K * U

    bf16 = jnp.bfloat16
    a = a_ref[...].reshape(LC, M)

    a2 = a.reshape(LC * (M // 128), 128).astype(bf16)
    i0 = jax.lax.broadcasted_iota(jnp.int32, (128, 128), 0)
    i1 = jax.lax.broadcasted_iota(jnp.int32, (128, 128), 1)
    bu = jnp.where(i0 // U == i1 // U, 1.0 / U, 0.0).astype(bf16)
    bk = jnp.where((i0 // KU == i1 // KU) & (i0 % U == i1 % U),
                   1.0 / K, 0.0).astype(bf16)
    mean_u = jnp.dot(a2, bu, preferred_element_type=f32).reshape(LC, M)
    mean_k = jnp.dot(a2, bk, preferred_element_type=f32).reshape(LC, M)

    p_all = jnp.concatenate(
        [p1_ref[...].reshape(LC2, C_in),
         0.1 * p2_ref[...].reshape(LC2, C_in),
         0.1 * p3_ref[...].reshape(LC2, C_in)], axis=1)
    c0 = jax.lax.broadcasted_iota(jnp.int32, (3 * C_in, 3 * LC), 0)
    c1 = jax.lax.broadcasted_iota(jnp.int32, (3 * C_in, 3 * LC), 1)
    sel = jnp.where(c0 == (c1 // LC) * C_in + c1 % C_in, 1.0, 0.0).astype(f32)
    w0 = jax.lax.broadcasted_iota(jnp.int32, (LC2, 3 * LC), 0)
    w1 = jax.lax.broadcasted_iota(jnp.int32, (LC2, 3 * LC), 1)
    mask = (w0 // C2 == (w1 % LC) // C_in).astype(f32)
    W = (jnp.dot(p_all, sel, preferred_element_type=f32) * mask).astype(bf16)

    cat = jnp.concatenate([a.astype(bf16), mean_u.astype(bf16),
                           mean_k.astype(bf16)], axis=0)
    y = jnp.dot(W, cat, preferred_element_type=f32)
    y = jnp.maximum(y, 0.0)

    f0 = jax.lax.broadcasted_iota(jnp.int32, (C2, LC2), 0)
    f1 = jax.lax.broadcasted_iota(jnp.int32, (C2, LC2), 1)
    fold = (f1 % C2 == f0).astype(bf16)
    yb = y.astype(bf16)
    sy = jnp.dot(fold, yb, preferred_element_type=f32)
    sz = jnp.dot(fold, yb * yb, preferred_element_type=f32)
    n = float(L * M)
    mu = sy.sum(axis=-1, keepdims=True) / n
    msq = sz.sum(axis=-1, keepdims=True) / n
    inv = jax.lax.rsqrt(msq - mu * mu + eps)
    y3 = y.reshape(L, C2, M)
    Mt = M // NT
    for i in range(NT):
        sl = slice(i * Mt, (i + 1) * Mt)
        ob_scr[:, :, sl] = (y3[:, :, sl] - mu[None, :, :]) * inv[None, :, :]
        pltpu.make_async_copy(ob_scr.at[:, :, sl], o_ref.at[:, :, sl],
                              sems.at[i]).start()
    for i in range(NT):
        sl = slice(i * Mt, (i + 1) * Mt)
        pltpu.make_async_copy(ob_scr.at[:, :, sl], o_ref.at[:, :, sl],
                              sems.at[i]).wait()


def kernel(A_lcm, P1, P2, P3, ru, bu, rk, bk):
    L, C_in, M = A_lcm.shape
    C2 = P1.shape[1]
    U = M // ru.shape[1]
    K = M // rk.shape[1]
    NT = 4 if M % (4 * 128) == 0 else 1
    return pl.pallas_call(
        functools.partial(_fused_body, K=K, U=U, NT=NT, eps=1e-5),
        out_shape=jax.ShapeDtypeStruct((L, C2, M), jnp.float32),
        out_specs=pl.BlockSpec(memory_space=pltpu.MemorySpace.HBM),
        scratch_shapes=[
            pltpu.VMEM((L, C2, M), jnp.float32),
            pltpu.SemaphoreType.DMA((NT,)),
        ],
        compiler_params=pltpu.CompilerParams(
            vmem_limit_bytes=48 << 20),
    )(A_lcm, P1, P2, P3)
